# baseline (speedup 1.0000x reference)
import functools

import jax
import jax.numpy as jnp
import numpy as np
from jax.experimental import pallas as pl
from jax.experimental.pallas import tpu as pltpu

B, D, H, O, E, K = 2048, 1024, 2048, 1024, 8, 2
EPS = float(np.finfo(np.float64).eps)

TM = 256
TMS = 256
A = B * K
NB = A // TMS + E
P = NB * TMS


def _router_body(x_ref, wg_ref, idx_ref, gate_ref):
    x = x_ref[...]
    wg = wg_ref[...]
    logits = jnp.dot(x, wg, preferred_element_type=jnp.float32)
    eidx = jax.lax.broadcasted_iota(jnp.int32, logits.shape, 1)
    v1 = jnp.max(logits, axis=1, keepdims=True)
    i1 = jnp.min(jnp.where(logits == v1, eidx, E), axis=1, keepdims=True)
    masked = jnp.where(eidx == i1, -jnp.inf, logits)
    v2 = jnp.max(masked, axis=1, keepdims=True)
    i2 = jnp.min(jnp.where(masked == v2, eidx, E), axis=1, keepdims=True)
    g1 = 1.0 / (1.0 + jnp.exp(v2 - v1))
    g2 = 1.0 / (1.0 + jnp.exp(v1 - v2))
    idx_ref[...] = jnp.concatenate([i1, i2], axis=1)
    gate_ref[...] = jnp.concatenate([g1, g2], axis=1)


def _group_body(be_ref, xg_ref, w1_ref, b1_ref, w2_ref, b2_ref, gate_ref,
                z_ref):
    xb = xg_ref[...].astype(jnp.bfloat16)
    h = jnp.dot(xb, w1_ref[0], preferred_element_type=jnp.float32)
    h = h + b1_ref[0]
    h = 0.5 * h * (1.0 + jax.lax.erf(h * np.float32(1.0 / np.sqrt(2.0))))
    out = jnp.dot(h.astype(jnp.bfloat16), w2_ref[0],
                  preferred_element_type=jnp.float32)
    out = out + b2_ref[0]
    m = jnp.max(out, axis=1, keepdims=True)
    p = jnp.exp(out - m)
    soft = p / jnp.sum(p, axis=1, keepdims=True)
    z_ref[...] = soft * gate_ref[...]


def _log_body(c_ref, o_ref):
    c = c_ref[...]
    c = jnp.where(c == 0.0, np.float32(EPS), c)
    o_ref[...] = jnp.log(c)


@jax.jit
def kernel(x, w_gate, fc1_w, fc1_b, fc2_w, fc2_b):
    nt = B // TM
    top_idx, top_gate = pl.pallas_call(
        _router_body,
        grid=(nt,),
        in_specs=[
            pl.BlockSpec((TM, D), lambda i: (i, 0)),
            pl.BlockSpec((D, E), lambda i: (0, 0)),
        ],
        out_specs=[
            pl.BlockSpec((TM, K), lambda i: (i, 0)),
            pl.BlockSpec((TM, K), lambda i: (i, 0)),
        ],
        out_shape=[
            jax.ShapeDtypeStruct((B, K), jnp.int32),
            jax.ShapeDtypeStruct((B, K), jnp.float32),
        ],
    )(x, w_gate)

    idx_flat = top_idx.reshape(-1)
    gate_flat = top_gate.reshape(-1)
    order = jnp.argsort(idx_flat, stable=True)
    counts = jnp.sum(idx_flat[None, :] == jnp.arange(E)[:, None], axis=1)
    blocks_e = (counts + TMS - 1) // TMS
    ends = jnp.cumsum(blocks_e)
    off = (ends - blocks_e) * TMS
    csum = jnp.cumsum(counts) - counts
    sorted_e = idx_flat[order]
    p_sorted = off[sorted_e] + (jnp.arange(A, dtype=jnp.int32) - csum[sorted_e])
    row_asgn = jnp.zeros((P,), jnp.int32).at[p_sorted].set(order)
    row_gate = jnp.zeros((P,), jnp.float32).at[p_sorted].set(gate_flat[order])
    pos = jnp.zeros((A,), jnp.int32).at[order].set(p_sorted)
    block_expert = jnp.minimum(
        jnp.searchsorted(ends, jnp.arange(NB), side="right"), E - 1
    ).astype(jnp.int32)

    xg = jnp.take(x, row_asgn // K, axis=0)

    w1 = fc1_w.astype(jnp.bfloat16)
    w2 = fc2_w.astype(jnp.bfloat16)
    z = pl.pallas_call(
        _group_body,
        grid_spec=pltpu.PrefetchScalarGridSpec(
            num_scalar_prefetch=1,
            grid=(NB,),
            in_specs=[
                pl.BlockSpec((TMS, D), lambda b, be: (b, 0)),
                pl.BlockSpec((1, D, H), lambda b, be: (be[b], 0, 0)),
                pl.BlockSpec((1, 1, H), lambda b, be: (be[b], 0, 0)),
                pl.BlockSpec((1, H, O), lambda b, be: (be[b], 0, 0)),
                pl.BlockSpec((1, 1, O), lambda b, be: (be[b], 0, 0)),
                pl.BlockSpec((TMS, 1), lambda b, be: (b, 0)),
            ],
            out_specs=pl.BlockSpec((TMS, O), lambda b, be: (b, 0)),
        ),
        out_shape=jax.ShapeDtypeStruct((P, O), jnp.float32),
        compiler_params=pltpu.CompilerParams(
            dimension_semantics=("arbitrary",)
        ),
    )(block_expert, xg, w1, fc1_b.reshape(E, 1, H), w2,
      fc2_b.reshape(E, 1, O), row_gate.reshape(P, 1))

    zc = jnp.take(z, pos.reshape(B, K), axis=0)
    combined = zc.sum(axis=1)

    out = pl.pallas_call(
        _log_body,
        grid=(nt,),
        in_specs=[pl.BlockSpec((TM, O), lambda i: (i, 0))],
        out_specs=pl.BlockSpec((TM, O), lambda i: (i, 0)),
        out_shape=jax.ShapeDtypeStruct((B, O), jnp.float32),
    )(combined)
    return out
